# baseline (speedup 1.0000x reference)
import functools

import jax
import jax.numpy as jnp
from jax.experimental import pallas as pl

BM = 400


def _dot(a, b):
    return jnp.dot(a, b, preferred_element_type=jnp.float32)


def _pass1_kernel(gso_ref, x_ref, t1_ref):
    t1_ref[...] = _dot(gso_ref[...], x_ref[...])


def _pass2_kernel(gso_ref, t1_ref, x_blk_ref, t1_blk_ref, w1_ref, b1_ref,
                  w2_ref, ab2_ref, d2_ref):
    t2 = _dot(gso_ref[...], t1_ref[...])
    x_blk = x_blk_ref[...]
    pre = (_dot(x_blk, w1_ref[0]) + _dot(t1_blk_ref[...], w1_ref[1])
           + _dot(2.0 * t2 - x_blk, w1_ref[2]) + b1_ref[...])
    h = jnp.maximum(pre, 0.0)
    ab2_ref[:, :64] = _dot(h, w2_ref[1])
    ab2_ref[:, 64:] = _dot(h, w2_ref[2])
    d2_ref[...] = _dot(h, w2_ref[0] - w2_ref[2])


def _pass3_kernel(gso_ref, ab2_ref, u2a_ref, u2b_ref):
    u2 = _dot(gso_ref[...], ab2_ref[...])
    u2a_ref[...] = u2[:, :64]
    u2b_ref[...] = u2[:, 64:]


def _pass4_kernel(gso_ref, u2b_ref, u2a_blk_ref, d2_blk_ref, b2_ref, out_ref):
    v2 = _dot(gso_ref[...], u2b_ref[...])
    logits = d2_blk_ref[...] + u2a_blk_ref[...] + 2.0 * v2 + b2_ref[...]
    m = jnp.max(logits, axis=1, keepdims=True)
    lse = jnp.log(jnp.sum(jnp.exp(logits - m), axis=1, keepdims=True)) + m
    out_ref[...] = logits - lse


@jax.jit
def kernel(x, gso, W1, b1, W2, b2):
    n, n_feat = x.shape
    n_hid = W1.shape[2]
    n_class = W2.shape[2]
    nb = n // BM
    b1r = b1.reshape(1, n_hid)
    b2r = b2.reshape(1, n_class)

    gso_spec = pl.BlockSpec((BM, n), lambda i: (i, 0))

    t1 = pl.pallas_call(
        _pass1_kernel,
        grid=(nb,),
        in_specs=[gso_spec, pl.BlockSpec((n, n_feat), lambda i: (0, 0))],
        out_specs=pl.BlockSpec((BM, n_feat), lambda i: (i, 0)),
        out_shape=jax.ShapeDtypeStruct((n, n_feat), jnp.float32),
    )(gso, x)

    ab2, d2 = pl.pallas_call(
        _pass2_kernel,
        grid=(nb,),
        in_specs=[
            gso_spec,
            pl.BlockSpec((n, n_feat), lambda i: (0, 0)),
            pl.BlockSpec((BM, n_feat), lambda i: (i, 0)),
            pl.BlockSpec((BM, n_feat), lambda i: (i, 0)),
            pl.BlockSpec((3, n_feat, n_hid), lambda i: (0, 0, 0)),
            pl.BlockSpec((1, n_hid), lambda i: (0, 0)),
            pl.BlockSpec((3, n_hid, n_class), lambda i: (0, 0, 0)),
        ],
        out_specs=[
            pl.BlockSpec((BM, 2 * n_class), lambda i: (i, 0)),
            pl.BlockSpec((BM, n_class), lambda i: (i, 0)),
        ],
        out_shape=[
            jax.ShapeDtypeStruct((n, 2 * n_class), jnp.float32),
            jax.ShapeDtypeStruct((n, n_class), jnp.float32),
        ],
    )(gso, t1, x, t1, W1, b1r, W2)

    u2a, u2b = pl.pallas_call(
        _pass3_kernel,
        grid=(nb,),
        in_specs=[gso_spec, pl.BlockSpec((n, 2 * n_class), lambda i: (0, 0))],
        out_specs=[
            pl.BlockSpec((BM, n_class), lambda i: (i, 0)),
            pl.BlockSpec((BM, n_class), lambda i: (i, 0)),
        ],
        out_shape=[
            jax.ShapeDtypeStruct((n, n_class), jnp.float32),
            jax.ShapeDtypeStruct((n, n_class), jnp.float32),
        ],
    )(gso, ab2)

    out = pl.pallas_call(
        _pass4_kernel,
        grid=(nb,),
        in_specs=[
            gso_spec,
            pl.BlockSpec((n, n_class), lambda i: (0, 0)),
            pl.BlockSpec((BM, n_class), lambda i: (i, 0)),
            pl.BlockSpec((BM, n_class), lambda i: (i, 0)),
            pl.BlockSpec((1, n_class), lambda i: (0, 0)),
        ],
        out_specs=pl.BlockSpec((BM, n_class), lambda i: (i, 0)),
        out_shape=jax.ShapeDtypeStruct((n, n_class), jnp.float32),
    )(gso, u2b, u2a, d2, b2r)

    return out
